# Initial kernel scaffold; baseline (speedup 1.0000x reference)
import jax
import jax.numpy as jnp
from jax.experimental import pallas as pl


def kernel(x, pos, edge_index, params):
    raise NotImplementedError("write your pallas kernel here")


# speedup vs baseline: 2.6360x; 2.6360x over previous
import functools

import jax
import jax.numpy as jnp
from jax import lax
from jax.experimental import pallas as pl
from jax.experimental.pallas import tpu as pltpu
from jax.experimental.pallas import tpu_sc as plsc

NC, NS, L = 2, 16, 16
NW = NC * NS

N = 50000
E = 800000
NPW = 1568
N_PAD = NW * NPW
CH = 8336
E_PAD = 96 * CH
FLUSH = 4000
BUF = FLUSH + 32
E_CAP = 804032
GC = 128
D = 64

_SC_PARAMS = pltpu.CompilerParams(needs_layout_passes=False,
                                  use_tc_tiling_on_sc=False)
_MESH = dict(core_axis_name="c", subcore_axis_name="s")


def _wid():
    return lax.axis_index("s") * NC + lax.axis_index("c")


def _filter_body(srcp_hbm, dstp_hbm, srcs_hbm, dls_hbm, cnts_hbm,
                 sin, din, sout, dlout, cbuf):
    wid = _wid()
    lo = wid * NPW

    def zinit(i, _):
        sout[pl.ds(i * 16, 16)] = jnp.zeros((16,), jnp.int32)
        return 0
    lax.fori_loop(0, BUF // 16, zinit, 0)

    ones = jnp.full((16,), 1, jnp.int32)
    zeros = jnp.zeros((16,), jnp.int32)

    def chunk(ci, carry):
        pltpu.sync_copy(srcp_hbm.at[pl.ds(ci * CH, CH)], sin)
        pltpu.sync_copy(dstp_hbm.at[pl.ds(ci * CH, CH)], din)

        def step(i, carry):
            cursor, total = carry
            sv = sin[pl.ds(i * 16, 16)]
            dv = din[pl.ds(i * 16, 16)]
            dl = dv - lo
            m = (dl >= 0) & (dl < NPW)
            plsc.store_compressed(sout.at[pl.ds(cursor, 16)], sv, mask=m)
            plsc.store_compressed(dlout.at[pl.ds(cursor, 16)], dl, mask=m)
            cursor = cursor + jnp.sum(jnp.where(m, ones, zeros))

            def do_flush(args):
                cur, tot = args
                tot8 = pl.multiple_of(tot, FLUSH)
                pltpu.sync_copy(sout.at[pl.ds(0, FLUSH)],
                                srcs_hbm.at[pl.ds(wid * E_CAP + tot8, FLUSH)])
                pltpu.sync_copy(dlout.at[pl.ds(0, FLUSH)],
                                dls_hbm.at[pl.ds(wid * E_CAP + tot8, FLUSH)])
                tv = sout[pl.ds(FLUSH, 16)]
                sout[pl.ds(0, 16)] = tv
                tv2 = dlout[pl.ds(FLUSH, 16)]
                dlout[pl.ds(0, 16)] = tv2
                return cur - FLUSH, tot + FLUSH

            cursor, total = lax.cond(cursor >= FLUSH, do_flush, lambda a: a,
                                     (cursor, total))
            return cursor, total

        return lax.fori_loop(0, CH // 16, step, carry)

    cursor, total = lax.fori_loop(0, E_PAD // CH, chunk, (0, 0))
    total8 = pl.multiple_of(total, FLUSH)
    pltpu.sync_copy(sout.at[pl.ds(0, BUF)],
                    srcs_hbm.at[pl.ds(wid * E_CAP + total8, BUF)])
    pltpu.sync_copy(dlout.at[pl.ds(0, BUF)],
                    dls_hbm.at[pl.ds(wid * E_CAP + total8, BUF)])
    cbuf[...] = jnp.zeros((16,), jnp.int32) + (total + cursor)
    pltpu.sync_copy(cbuf, cnts_hbm.at[pl.ds(wid * 16, 16)])


def _build_edge_lists(srcp, dstp):
    f = functools.partial(
        pl.kernel,
        mesh=plsc.VectorSubcoreMesh(**_MESH),
        out_type=(jax.ShapeDtypeStruct((NW * E_CAP,), jnp.int32),
                  jax.ShapeDtypeStruct((NW * E_CAP,), jnp.int32),
                  jax.ShapeDtypeStruct((NW * 16,), jnp.int32)),
        scratch_types=[
            pltpu.VMEM((CH,), jnp.int32),
            pltpu.VMEM((CH,), jnp.int32),
            pltpu.VMEM((BUF,), jnp.int32),
            pltpu.VMEM((BUF,), jnp.int32),
            pltpu.VMEM((16,), jnp.int32),
        ],
        compiler_params=_SC_PARAMS,
    )(_filter_body)
    return f(srcp, dstp)


def _segmax_body(a_hbm, srcs_hbm, dls_hbm, cnts_hbm, s_hbm,
                 acc, idxv, dlv, rows, cbuf, sem):
    wid = _wid()
    neg = jnp.full((16,), -jnp.inf, jnp.float32)

    def init(i, _):
        acc[pl.ds(i * 16, 16)] = neg
        return 0
    lax.fori_loop(0, NPW * D // 16, init, 0)

    pltpu.sync_copy(cnts_hbm.at[pl.ds(wid * 16, 16)], cbuf)
    cnt = cbuf[pl.ds(0, 16)][0]
    nch = (cnt + (GC - 1)) // GC

    def chunk(c, _):
        off = wid * E_CAP + c * GC
        pltpu.sync_copy(srcs_hbm.at[pl.ds(off, GC)], idxv)
        pltpu.sync_copy(dls_hbm.at[pl.ds(off, GC)], dlv.at[pl.ds(0, GC)])
        pltpu.async_copy(a_hbm.at[idxv], rows, sem).wait()
        clen = jnp.minimum(cnt - c * GC, GC)

        def edge(e, _):
            d = dlv[pl.ds(e, 16)][0]
            o = d * D
            for fb in range(D // 16):
                sl = pl.ds(o + fb * 16, 16)
                acc[sl] = jnp.maximum(acc[sl], rows[e, pl.ds(fb * 16, 16)])
            return 0
        lax.fori_loop(0, clen, edge, 0)
        return 0

    lax.fori_loop(0, nch, chunk, 0)
    pltpu.sync_copy(acc, s_hbm.at[pl.ds(wid * NPW * D, NPW * D)])


def _segment_max(a, srcs, dls, cnts):
    f = functools.partial(
        pl.kernel,
        mesh=plsc.VectorSubcoreMesh(**_MESH),
        out_type=jax.ShapeDtypeStruct((N_PAD * D,), jnp.float32),
        scratch_types=[
            pltpu.VMEM((NPW * D,), jnp.float32),
            pltpu.VMEM((GC,), jnp.int32),
            pltpu.VMEM((GC + 16,), jnp.int32),
            pltpu.VMEM((GC, D), jnp.float32),
            pltpu.VMEM((16,), jnp.int32),
            pltpu.SemaphoreType.DMA,
        ],
        compiler_params=_SC_PARAMS,
    )(_segmax_body)
    return f(a, srcs, dls, cnts).reshape(N_PAD, D)


BLK = 512
GRID = N_PAD // BLK


def _pos_mm(p, wp):
    return (p[:, 0:1] * wp[0:1, :] + p[:, 1:2] * wp[1:2, :]
            + p[:, 2:3] * wp[2:3, :])


def _ln(h, g, b):
    mu = jnp.mean(h, axis=-1, keepdims=True)
    var = jnp.mean((h - mu) ** 2, axis=-1, keepdims=True)
    return (h - mu) * lax.rsqrt(var + 1e-5) * g + b


def _row_spec(w=D):
    return pl.BlockSpec((BLK, w), lambda i: (i, 0))


def _full_spec(shape):
    return pl.BlockSpec(shape, lambda i: tuple(0 for _ in shape))


def _tc_first_body(x_ref, pos_ref, wh_ref, wp_ref, a_ref):
    a_ref[...] = x_ref[...] * wh_ref[...] + _pos_mm(pos_ref[...], wp_ref[...])


def _tc_first(xp, posp, wh0, wp0):
    return pl.pallas_call(
        _tc_first_body,
        grid=(GRID,),
        in_specs=[_row_spec(1), _row_spec(3), _full_spec((1, D)),
                  _full_spec((3, D))],
        out_specs=_row_spec(),
        out_shape=jax.ShapeDtypeStruct((N_PAD, D), jnp.float32),
    )(xp, posp, wh0, wp0)


def _tc_mid_body(s_ref, pos_ref, wpp_ref, g_ref, b_ref, wh_ref, wp_ref,
                 emb_ref, a_ref):
    p = pos_ref[...]
    s = s_ref[...]
    bb = _pos_mm(p, wpp_ref[...])
    o = jnp.where(s != -jnp.inf, s - bb, 0.0)
    emb = jax.nn.relu(_ln(o, g_ref[...], b_ref[...]))
    emb_ref[...] = emb
    a_ref[...] = (jnp.dot(emb, wh_ref[...], preferred_element_type=jnp.float32)
                  + _pos_mm(p, wp_ref[...]))


def _tc_mid(s, posp, wp_prev, g, b, wh, wp):
    return pl.pallas_call(
        _tc_mid_body,
        grid=(GRID,),
        in_specs=[_row_spec(), _row_spec(3), _full_spec((3, D)),
                  _full_spec((1, D)), _full_spec((1, D)),
                  _full_spec((D, D)), _full_spec((3, D))],
        out_specs=(_row_spec(), _row_spec()),
        out_shape=(jax.ShapeDtypeStruct((N_PAD, D), jnp.float32),
                   jax.ShapeDtypeStruct((N_PAD, D), jnp.float32)),
    )(s, posp, wp_prev, g, b, wh, wp)


def _tc_final_body(s_ref, pos_ref, wpp_ref, g_ref, b_ref,
                   e0_ref, e1_ref, e2_ref, e3_ref,
                   w0_ref, b0_ref, w1_ref, b1_ref, w2_ref, b2_ref,
                   w3_ref, b3_ref, sc_ref, out_ref):
    s = s_ref[...]
    bb = _pos_mm(pos_ref[...], wpp_ref[...])
    o = jnp.where(s != -jnp.inf, s - bb, 0.0)
    emb4 = jax.nn.relu(_ln(o, g_ref[...], b_ref[...]))
    z = jnp.concatenate(
        [e0_ref[...], e1_ref[...], e2_ref[...], e3_ref[...], emb4], axis=-1)
    mu = jnp.mean(z, axis=-1, keepdims=True)
    var = jnp.mean((z - mu) ** 2, axis=-1, keepdims=True)
    z = (z - mu) * lax.rsqrt(var + 1e-5)
    z = jax.nn.relu(jnp.dot(z, w0_ref[...], preferred_element_type=jnp.float32)
                    + b0_ref[...])
    z = jax.nn.relu(jnp.dot(z, w1_ref[...], preferred_element_type=jnp.float32)
                    + b1_ref[...])
    z = jax.nn.relu(jnp.dot(z, w2_ref[...], preferred_element_type=jnp.float32)
                    + b2_ref[...])
    z = (jnp.dot(z, w3_ref[...], preferred_element_type=jnp.float32)
         + b3_ref[...])
    out_ref[...] = z * sc_ref[...]


def _tc_final(s4, posp, wp4, g4, b4, embs, mw, mb, scale):
    return pl.pallas_call(
        _tc_final_body,
        grid=(GRID,),
        in_specs=[_row_spec(), _row_spec(3), _full_spec((3, D)),
                  _full_spec((1, D)), _full_spec((1, D)),
                  _row_spec(), _row_spec(), _row_spec(), _row_spec(),
                  _full_spec((320, 128)), _full_spec((1, 128)),
                  _full_spec((128, 128)), _full_spec((1, 128)),
                  _full_spec((128, 64)), _full_spec((1, 64)),
                  _full_spec((64, 2)), _full_spec((1, 2)),
                  _full_spec((1, 2))],
        out_specs=_row_spec(2),
        out_shape=jax.ShapeDtypeStruct((N_PAD, 2), jnp.float32),
    )(s4, posp, wp4, g4, b4, embs[0], embs[1], embs[2], embs[3],
      mw[0], mb[0], mw[1], mb[1], mw[2], mb[2], mw[3], mb[3], scale)


def kernel(x, pos, edge_index, params):
    src = edge_index[0]
    dst = edge_index[1]
    srcp = jnp.concatenate([src, jnp.zeros((E_PAD - E,), jnp.int32)])
    dstp = jnp.concatenate([dst, jnp.full((E_PAD - E,), 2 * N_PAD, jnp.int32)])
    xp = jnp.pad(x, ((0, N_PAD - N), (0, 0)))
    posp = jnp.pad(pos, ((0, N_PAD - N), (0, 0)))

    srcs, dls, cnts = _build_edge_lists(srcp, dstp)

    ws = [params["w%d" % i] for i in range(5)]
    whs = [ws[0][:1]] + [w[:D] for w in ws[1:]]
    wps = [w[-3:] for w in ws]
    lgs = [params["ln_g%d" % i].reshape(1, D) for i in range(5)]
    lbs = [params["ln_b%d" % i].reshape(1, D) for i in range(5)]

    a = _tc_first(xp, posp, whs[0], wps[0])
    s = _segment_max(a, srcs, dls, cnts)
    embs = []
    for i in range(1, 5):
        emb, a = _tc_mid(s, posp, wps[i - 1], lgs[i - 1], lbs[i - 1],
                         whs[i], wps[i])
        embs.append(emb)
        s = _segment_max(a, srcs, dls, cnts)

    mw = [params["mlp_w%d" % j] for j in range(4)]
    mb = [params["mlp_b%d" % j].reshape(1, -1) for j in range(4)]
    out = _tc_final(s, posp, wps[4], lgs[4], lbs[4], embs, mw, mb,
                    params["scale"].reshape(1, 2))
    return out[:N]


# speedup vs baseline: 2.6761x; 1.0152x over previous
import functools

import jax
import jax.numpy as jnp
from jax import lax
from jax.experimental import pallas as pl
from jax.experimental.pallas import tpu as pltpu
from jax.experimental.pallas import tpu_sc as plsc

NC, NS, L = 2, 16, 16
NW = NC * NS

N = 50000
E = 800000
NPW = 1568
N_PAD = NW * NPW
CH = 8336
E_PAD = 96 * CH
E_IN = E_PAD + CH
FLUSH = 4000
BUF = FLUSH + 32
IB = 4096
E_CAP = 804352
GC = 128
D = 64

_SC_PARAMS = pltpu.CompilerParams(needs_layout_passes=False,
                                  use_tc_tiling_on_sc=False)
_MESH = dict(core_axis_name="c", subcore_axis_name="s")


def _wid():
    return lax.axis_index("s") * NC + lax.axis_index("c")


def _filter_body(srcp_hbm, dstp_hbm, srcs_hbm, dls_hbm, cnts_hbm,
                 sin0, din0, sin1, din1, sout, dlout, cbuf,
                 ss0, sd0, ss1, sd1):
    wid = _wid()
    lo = wid * NPW

    def zinit(i, _):
        sout[pl.ds(i * 16, 16)] = jnp.zeros((16,), jnp.int32)
        return 0
    lax.fori_loop(0, BUF // 16, zinit, 0)

    def start(ci, sb, db, ss, sd):
        pltpu.async_copy(srcp_hbm.at[pl.ds(ci * CH, CH)], sb, ss)
        pltpu.async_copy(dstp_hbm.at[pl.ds(ci * CH, CH)], db, sd)

    def wait_in(sb, db, ss, sd):
        pltpu.make_async_copy(srcp_hbm.at[pl.ds(0, CH)], sb, ss).wait()
        pltpu.make_async_copy(dstp_hbm.at[pl.ds(0, CH)], db, sd).wait()

    def process(sin, din, carry):
        def step(i, carry):
            cursor, total = carry
            sv = sin[pl.ds(i * 16, 16)]
            dv = din[pl.ds(i * 16, 16)]
            dl = dv - lo
            m = (dl >= 0) & (dl < NPW)
            plsc.store_compressed(sout.at[pl.ds(cursor, 16)], sv, mask=m)
            plsc.store_compressed(dlout.at[pl.ds(cursor, 16)], dl, mask=m)
            cursor = cursor + plsc.all_reduce_population_count(m)[0]

            def do_flush(args):
                cur, tot = args
                tot8 = pl.multiple_of(tot, FLUSH)
                pltpu.sync_copy(sout.at[pl.ds(0, FLUSH)],
                                srcs_hbm.at[pl.ds(wid * E_CAP + tot8, FLUSH)])
                pltpu.sync_copy(dlout.at[pl.ds(0, FLUSH)],
                                dls_hbm.at[pl.ds(wid * E_CAP + tot8, FLUSH)])
                tv = sout[pl.ds(FLUSH, 16)]
                sout[pl.ds(0, 16)] = tv
                tv2 = dlout[pl.ds(FLUSH, 16)]
                dlout[pl.ds(0, 16)] = tv2
                return cur - FLUSH, tot + FLUSH

            return lax.cond(cursor >= FLUSH, do_flush, lambda a: a,
                            (cursor, total))

        return lax.fori_loop(0, CH // 16, step, carry)

    start(0, sin0, din0, ss0, sd0)

    def super_step(i, carry):
        wait_in(sin0, din0, ss0, sd0)
        start(2 * i + 1, sin1, din1, ss1, sd1)
        carry = process(sin0, din0, carry)
        wait_in(sin1, din1, ss1, sd1)
        start(2 * i + 2, sin0, din0, ss0, sd0)
        carry = process(sin1, din1, carry)
        return carry

    cursor, total = lax.fori_loop(0, E_PAD // (2 * CH), super_step, (0, 0))
    wait_in(sin0, din0, ss0, sd0)
    total8 = pl.multiple_of(total, FLUSH)
    pltpu.sync_copy(sout.at[pl.ds(0, BUF)],
                    srcs_hbm.at[pl.ds(wid * E_CAP + total8, BUF)])
    pltpu.sync_copy(dlout.at[pl.ds(0, BUF)],
                    dls_hbm.at[pl.ds(wid * E_CAP + total8, BUF)])
    cbuf[...] = jnp.zeros((16,), jnp.int32) + (total + cursor)
    pltpu.sync_copy(cbuf, cnts_hbm.at[pl.ds(wid * 16, 16)])


def _build_edge_lists(srcp, dstp):
    f = functools.partial(
        pl.kernel,
        mesh=plsc.VectorSubcoreMesh(**_MESH),
        out_type=(jax.ShapeDtypeStruct((NW * E_CAP,), jnp.int32),
                  jax.ShapeDtypeStruct((NW * E_CAP,), jnp.int32),
                  jax.ShapeDtypeStruct((NW * 16,), jnp.int32)),
        scratch_types=[
            pltpu.VMEM((CH,), jnp.int32),
            pltpu.VMEM((CH,), jnp.int32),
            pltpu.VMEM((CH,), jnp.int32),
            pltpu.VMEM((CH,), jnp.int32),
            pltpu.VMEM((BUF,), jnp.int32),
            pltpu.VMEM((BUF,), jnp.int32),
            pltpu.VMEM((16,), jnp.int32),
            pltpu.SemaphoreType.DMA,
            pltpu.SemaphoreType.DMA,
            pltpu.SemaphoreType.DMA,
            pltpu.SemaphoreType.DMA,
        ],
        compiler_params=_SC_PARAMS,
    )(_filter_body)
    return f(srcp, dstp)


NCHB = IB // GC


def _segmax_body(a_hbm, srcs_hbm, dls_hbm, cnts_hbm, s_hbm,
                 acc, idxv, dlv, rows, cbuf, sem):
    wid = _wid()
    neg = jnp.full((16,), -jnp.inf, jnp.float32)

    def init(i, _):
        acc[pl.ds(i * 16, 16)] = neg
        return 0
    lax.fori_loop(0, NPW * D // 16, init, 0)

    pltpu.sync_copy(cnts_hbm.at[pl.ds(wid * 16, 16)], cbuf)
    cnt = cbuf[pl.ds(0, 16)][0]
    nch = (cnt + (GC - 1)) // GC

    def chunk(c, _):
        off = wid * E_CAP + c * GC
        pltpu.sync_copy(srcs_hbm.at[pl.ds(off, GC)], idxv)
        pltpu.sync_copy(dls_hbm.at[pl.ds(off, GC)], dlv.at[pl.ds(0, GC)])
        pltpu.async_copy(a_hbm.at[idxv], rows, sem).wait()
        clen = jnp.minimum(cnt - c * GC, GC)

        def edge(e, _):
            d = dlv[pl.ds(e, 16)][0]
            o = d * D
            for fb in range(D // 16):
                sl = pl.ds(o + fb * 16, 16)
                acc[sl] = jnp.maximum(acc[sl], rows[e, pl.ds(fb * 16, 16)])
            return 0
        lax.fori_loop(0, clen, edge, 0)
        return 0

    lax.fori_loop(0, nch, chunk, 0)
    pltpu.sync_copy(acc, s_hbm.at[pl.ds(wid * NPW * D, NPW * D)])


def _segment_max(a, srcs, dls, cnts):
    f = functools.partial(
        pl.kernel,
        mesh=plsc.VectorSubcoreMesh(**_MESH),
        out_type=jax.ShapeDtypeStruct((N_PAD * D,), jnp.float32),
        scratch_types=[
            pltpu.VMEM((NPW * D,), jnp.float32),
            pltpu.VMEM((GC,), jnp.int32),
            pltpu.VMEM((GC + 16,), jnp.int32),
            pltpu.VMEM((GC, D), jnp.float32),
            pltpu.VMEM((16,), jnp.int32),
            pltpu.SemaphoreType.DMA,
        ],
        compiler_params=_SC_PARAMS,
    )(_segmax_body)
    return f(a, srcs, dls, cnts).reshape(N_PAD, D)


BLK = 512
GRID = N_PAD // BLK


def _pos_mm(p, wp):
    return (p[:, 0:1] * wp[0:1, :] + p[:, 1:2] * wp[1:2, :]
            + p[:, 2:3] * wp[2:3, :])


def _ln(h, g, b):
    mu = jnp.mean(h, axis=-1, keepdims=True)
    var = jnp.mean((h - mu) ** 2, axis=-1, keepdims=True)
    return (h - mu) * lax.rsqrt(var + 1e-5) * g + b


def _row_spec(w=D):
    return pl.BlockSpec((BLK, w), lambda i: (i, 0))


def _full_spec(shape):
    return pl.BlockSpec(shape, lambda i: tuple(0 for _ in shape))


def _tc_first_body(x_ref, pos_ref, wh_ref, wp_ref, a_ref):
    a_ref[...] = x_ref[...] * wh_ref[...] + _pos_mm(pos_ref[...], wp_ref[...])


def _tc_first(xp, posp, wh0, wp0):
    return pl.pallas_call(
        _tc_first_body,
        grid=(GRID,),
        in_specs=[_row_spec(1), _row_spec(3), _full_spec((1, D)),
                  _full_spec((3, D))],
        out_specs=_row_spec(),
        out_shape=jax.ShapeDtypeStruct((N_PAD, D), jnp.float32),
    )(xp, posp, wh0, wp0)


def _tc_mid_body(s_ref, pos_ref, wpp_ref, g_ref, b_ref, wh_ref, wp_ref,
                 emb_ref, a_ref):
    p = pos_ref[...]
    s = s_ref[...]
    bb = _pos_mm(p, wpp_ref[...])
    o = jnp.where(s != -jnp.inf, s - bb, 0.0)
    emb = jax.nn.relu(_ln(o, g_ref[...], b_ref[...]))
    emb_ref[...] = emb
    a_ref[...] = (jnp.dot(emb, wh_ref[...], preferred_element_type=jnp.float32, precision=lax.Precision.HIGHEST)
                  + _pos_mm(p, wp_ref[...]))


def _tc_mid(s, posp, wp_prev, g, b, wh, wp):
    return pl.pallas_call(
        _tc_mid_body,
        grid=(GRID,),
        in_specs=[_row_spec(), _row_spec(3), _full_spec((3, D)),
                  _full_spec((1, D)), _full_spec((1, D)),
                  _full_spec((D, D)), _full_spec((3, D))],
        out_specs=(_row_spec(), _row_spec()),
        out_shape=(jax.ShapeDtypeStruct((N_PAD, D), jnp.float32),
                   jax.ShapeDtypeStruct((N_PAD, D), jnp.float32)),
    )(s, posp, wp_prev, g, b, wh, wp)


def _tc_final_body(s_ref, pos_ref, wpp_ref, g_ref, b_ref,
                   e0_ref, e1_ref, e2_ref, e3_ref,
                   w0_ref, b0_ref, w1_ref, b1_ref, w2_ref, b2_ref,
                   w3_ref, b3_ref, sc_ref, out_ref):
    s = s_ref[...]
    bb = _pos_mm(pos_ref[...], wpp_ref[...])
    o = jnp.where(s != -jnp.inf, s - bb, 0.0)
    emb4 = jax.nn.relu(_ln(o, g_ref[...], b_ref[...]))
    z = jnp.concatenate(
        [e0_ref[...], e1_ref[...], e2_ref[...], e3_ref[...], emb4], axis=-1)
    mu = jnp.mean(z, axis=-1, keepdims=True)
    var = jnp.mean((z - mu) ** 2, axis=-1, keepdims=True)
    z = (z - mu) * lax.rsqrt(var + 1e-5)
    z = jax.nn.relu(jnp.dot(z, w0_ref[...], preferred_element_type=jnp.float32, precision=lax.Precision.HIGHEST)
                    + b0_ref[...])
    z = jax.nn.relu(jnp.dot(z, w1_ref[...], preferred_element_type=jnp.float32, precision=lax.Precision.HIGHEST)
                    + b1_ref[...])
    z = jax.nn.relu(jnp.dot(z, w2_ref[...], preferred_element_type=jnp.float32, precision=lax.Precision.HIGHEST)
                    + b2_ref[...])
    z = (jnp.dot(z, w3_ref[...], preferred_element_type=jnp.float32, precision=lax.Precision.HIGHEST)
         + b3_ref[...])
    out_ref[...] = z * sc_ref[...]


def _tc_final(s4, posp, wp4, g4, b4, embs, mw, mb, scale):
    return pl.pallas_call(
        _tc_final_body,
        grid=(GRID,),
        in_specs=[_row_spec(), _row_spec(3), _full_spec((3, D)),
                  _full_spec((1, D)), _full_spec((1, D)),
                  _row_spec(), _row_spec(), _row_spec(), _row_spec(),
                  _full_spec((320, 128)), _full_spec((1, 128)),
                  _full_spec((128, 128)), _full_spec((1, 128)),
                  _full_spec((128, 64)), _full_spec((1, 64)),
                  _full_spec((64, 2)), _full_spec((1, 2)),
                  _full_spec((1, 2))],
        out_specs=_row_spec(2),
        out_shape=jax.ShapeDtypeStruct((N_PAD, 2), jnp.float32),
    )(s4, posp, wp4, g4, b4, embs[0], embs[1], embs[2], embs[3],
      mw[0], mb[0], mw[1], mb[1], mw[2], mb[2], mw[3], mb[3], scale)


def kernel(x, pos, edge_index, params):
    src = edge_index[0]
    dst = edge_index[1]
    srcp = jnp.concatenate([src, jnp.zeros((E_IN - E,), jnp.int32)])
    dstp = jnp.concatenate([dst, jnp.full((E_IN - E,), 2 * N_PAD, jnp.int32)])
    xp = jnp.pad(x, ((0, N_PAD - N), (0, 0)))
    posp = jnp.pad(pos, ((0, N_PAD - N), (0, 0)))

    srcs, dls, cnts = _build_edge_lists(srcp, dstp)

    ws = [params["w%d" % i] for i in range(5)]
    whs = [ws[0][:1]] + [w[:D] for w in ws[1:]]
    wps = [w[-3:] for w in ws]
    lgs = [params["ln_g%d" % i].reshape(1, D) for i in range(5)]
    lbs = [params["ln_b%d" % i].reshape(1, D) for i in range(5)]

    a = _tc_first(xp, posp, whs[0], wps[0])
    s = _segment_max(a, srcs, dls, cnts)
    embs = []
    for i in range(1, 5):
        emb, a = _tc_mid(s, posp, wps[i - 1], lgs[i - 1], lbs[i - 1],
                         whs[i], wps[i])
        embs.append(emb)
        s = _segment_max(a, srcs, dls, cnts)

    mw = [params["mlp_w%d" % j] for j in range(4)]
    mb = [params["mlp_b%d" % j].reshape(1, -1) for j in range(4)]
    out = _tc_final(s, posp, wps[4], lgs[4], lbs[4], embs, mw, mb,
                    params["scale"].reshape(1, 2))
    return out[:N]


# speedup vs baseline: 2.7531x; 1.0288x over previous
import functools

import jax
import jax.numpy as jnp
from jax import lax
from jax.experimental import pallas as pl
from jax.experimental.pallas import tpu as pltpu
from jax.experimental.pallas import tpu_sc as plsc

NC, NS, L = 2, 16, 16
NW = NC * NS

N = 50000
E = 800000
NPW = 1568
N_PAD = NW * NPW
CH = 8336
E_PAD = 96 * CH
E_IN = E_PAD + CH
FLUSH = 4000
BUF = FLUSH + 32
IB = 4096
E_CAP = 804352
GC = 128
D = 64

_SC_PARAMS = pltpu.CompilerParams(needs_layout_passes=False,
                                  use_tc_tiling_on_sc=False)
_MESH = dict(core_axis_name="c", subcore_axis_name="s")


def _wid():
    return lax.axis_index("s") * NC + lax.axis_index("c")


def _filter_body(srcp_hbm, dstp_hbm, srcs_hbm, dls_hbm, cnts_hbm,
                 sin0, din0, sin1, din1, sout, dlout, cbuf,
                 ss0, sd0, ss1, sd1):
    wid = _wid()
    lo = wid * NPW

    def zinit(i, _):
        sout[pl.ds(i * 16, 16)] = jnp.zeros((16,), jnp.int32)
        return 0
    lax.fori_loop(0, BUF // 16, zinit, 0)

    def start(ci, sb, db, ss, sd):
        pltpu.async_copy(srcp_hbm.at[pl.ds(ci * CH, CH)], sb, ss)
        pltpu.async_copy(dstp_hbm.at[pl.ds(ci * CH, CH)], db, sd)

    def wait_in(sb, db, ss, sd):
        pltpu.make_async_copy(srcp_hbm.at[pl.ds(0, CH)], sb, ss).wait()
        pltpu.make_async_copy(dstp_hbm.at[pl.ds(0, CH)], db, sd).wait()

    def process(sin, din, carry):
        def step(i, carry):
            cursor, total = carry
            sv = sin[pl.ds(i * 16, 16)]
            dv = din[pl.ds(i * 16, 16)]
            dl = dv - lo
            m = (dl >= 0) & (dl < NPW)
            plsc.store_compressed(sout.at[pl.ds(cursor, 16)], sv, mask=m)
            plsc.store_compressed(dlout.at[pl.ds(cursor, 16)], dl, mask=m)
            cursor = cursor + plsc.all_reduce_population_count(m)[0]

            def do_flush(args):
                cur, tot = args
                tot8 = pl.multiple_of(tot, FLUSH)
                pltpu.sync_copy(sout.at[pl.ds(0, FLUSH)],
                                srcs_hbm.at[pl.ds(wid * E_CAP + tot8, FLUSH)])
                pltpu.sync_copy(dlout.at[pl.ds(0, FLUSH)],
                                dls_hbm.at[pl.ds(wid * E_CAP + tot8, FLUSH)])
                tv = sout[pl.ds(FLUSH, 16)]
                sout[pl.ds(0, 16)] = tv
                tv2 = dlout[pl.ds(FLUSH, 16)]
                dlout[pl.ds(0, 16)] = tv2
                return cur - FLUSH, tot + FLUSH

            return lax.cond(cursor >= FLUSH, do_flush, lambda a: a,
                            (cursor, total))

        return lax.fori_loop(0, CH // 16, step, carry)

    start(0, sin0, din0, ss0, sd0)

    def super_step(i, carry):
        wait_in(sin0, din0, ss0, sd0)
        start(2 * i + 1, sin1, din1, ss1, sd1)
        carry = process(sin0, din0, carry)
        wait_in(sin1, din1, ss1, sd1)
        start(2 * i + 2, sin0, din0, ss0, sd0)
        carry = process(sin1, din1, carry)
        return carry

    cursor, total = lax.fori_loop(0, E_PAD // (2 * CH), super_step, (0, 0))
    wait_in(sin0, din0, ss0, sd0)
    total8 = pl.multiple_of(total, FLUSH)
    pltpu.sync_copy(sout.at[pl.ds(0, BUF)],
                    srcs_hbm.at[pl.ds(wid * E_CAP + total8, BUF)])
    pltpu.sync_copy(dlout.at[pl.ds(0, BUF)],
                    dls_hbm.at[pl.ds(wid * E_CAP + total8, BUF)])
    cbuf[...] = jnp.zeros((16,), jnp.int32) + (total + cursor)
    pltpu.sync_copy(cbuf, cnts_hbm.at[pl.ds(wid * 16, 16)])


def _build_edge_lists(srcp, dstp):
    f = functools.partial(
        pl.kernel,
        mesh=plsc.VectorSubcoreMesh(**_MESH),
        out_type=(jax.ShapeDtypeStruct((NW * E_CAP,), jnp.int32),
                  jax.ShapeDtypeStruct((NW * E_CAP,), jnp.int32),
                  jax.ShapeDtypeStruct((NW * 16,), jnp.int32)),
        scratch_types=[
            pltpu.VMEM((CH,), jnp.int32),
            pltpu.VMEM((CH,), jnp.int32),
            pltpu.VMEM((CH,), jnp.int32),
            pltpu.VMEM((CH,), jnp.int32),
            pltpu.VMEM((BUF,), jnp.int32),
            pltpu.VMEM((BUF,), jnp.int32),
            pltpu.VMEM((16,), jnp.int32),
            pltpu.SemaphoreType.DMA,
            pltpu.SemaphoreType.DMA,
            pltpu.SemaphoreType.DMA,
            pltpu.SemaphoreType.DMA,
        ],
        compiler_params=_SC_PARAMS,
    )(_filter_body)
    return f(srcp, dstp)


NCHB = IB // GC


def _segmax_body(a_hbm, srcs_hbm, dls_hbm, cnts_hbm, s_hbm,
                 acc, idxv, dlv, rows, cbuf, sem):
    wid = _wid()
    neg = jnp.full((16,), -jnp.inf, jnp.float32)

    def init(i, _):
        acc[pl.ds(i * 16, 16)] = neg
        return 0
    lax.fori_loop(0, NPW * D // 16, init, 0)

    pltpu.sync_copy(cnts_hbm.at[pl.ds(wid * 16, 16)], cbuf)
    cnt = cbuf[pl.ds(0, 16)][0]
    nch = (cnt + (GC - 1)) // GC

    def chunk(c, _):
        off = wid * E_CAP + c * GC
        pltpu.sync_copy(srcs_hbm.at[pl.ds(off, GC)], idxv)
        pltpu.sync_copy(dls_hbm.at[pl.ds(off, GC)], dlv.at[pl.ds(0, GC)])
        pltpu.async_copy(a_hbm.at[idxv], rows, sem).wait()
        clen = jnp.minimum(cnt - c * GC, GC)

        def edge(e, _):
            d = dlv[pl.ds(e, 16)][0]
            o = d * D
            for fb in range(D // 16):
                sl = pl.ds(o + fb * 16, 16)
                acc[sl] = jnp.maximum(acc[sl], rows[e, pl.ds(fb * 16, 16)])
            return 0
        lax.fori_loop(0, clen, edge, 0)
        return 0

    lax.fori_loop(0, nch, chunk, 0)
    pltpu.sync_copy(acc, s_hbm.at[pl.ds(wid * NPW * D, NPW * D)])


def _segment_max(a, srcs, dls, cnts):
    f = functools.partial(
        pl.kernel,
        mesh=plsc.VectorSubcoreMesh(**_MESH),
        out_type=jax.ShapeDtypeStruct((N_PAD * D,), jnp.float32),
        scratch_types=[
            pltpu.VMEM((NPW * D,), jnp.float32),
            pltpu.VMEM((GC,), jnp.int32),
            pltpu.VMEM((GC + 16,), jnp.int32),
            pltpu.VMEM((GC, D), jnp.float32),
            pltpu.VMEM((16,), jnp.int32),
            pltpu.SemaphoreType.DMA,
        ],
        compiler_params=_SC_PARAMS,
    )(_segmax_body)
    return f(a, srcs, dls, cnts).reshape(N_PAD, D)


BLK = 512
GRID = N_PAD // BLK


def _pos_mm(p, wp):
    return (p[:, 0:1] * wp[0:1, :] + p[:, 1:2] * wp[1:2, :]
            + p[:, 2:3] * wp[2:3, :])


def _ln(h, g, b):
    mu = jnp.mean(h, axis=-1, keepdims=True)
    var = jnp.mean((h - mu) ** 2, axis=-1, keepdims=True)
    return (h - mu) * lax.rsqrt(var + 1e-5) * g + b


def _row_spec(w=D):
    return pl.BlockSpec((BLK, w), lambda i: (i, 0))


def _full_spec(shape):
    return pl.BlockSpec(shape, lambda i: tuple(0 for _ in shape))


def _tc_first_body(x_ref, pos_ref, wh_ref, wp_ref, a_ref):
    a_ref[...] = x_ref[...] * wh_ref[...] + _pos_mm(pos_ref[...], wp_ref[...])


def _tc_first(xp, posp, wh0, wp0):
    return pl.pallas_call(
        _tc_first_body,
        grid=(GRID,),
        in_specs=[_row_spec(1), _row_spec(3), _full_spec((1, D)),
                  _full_spec((3, D))],
        out_specs=_row_spec(),
        out_shape=jax.ShapeDtypeStruct((N_PAD, D), jnp.float32),
    )(xp, posp, wh0, wp0)


def _tc_mid_body(s_ref, pos_ref, wpp_ref, g_ref, b_ref, wh_ref, wp_ref,
                 emb_ref, a_ref):
    p = pos_ref[...]
    s = s_ref[...]
    bb = _pos_mm(p, wpp_ref[...])
    o = jnp.where(s != -jnp.inf, s - bb, 0.0)
    emb = jax.nn.relu(_ln(o, g_ref[...], b_ref[...]))
    emb_ref[...] = emb
    a_ref[...] = (jnp.dot(emb, wh_ref[...], preferred_element_type=jnp.float32)
                  + _pos_mm(p, wp_ref[...]))


def _tc_mid(s, posp, wp_prev, g, b, wh, wp):
    return pl.pallas_call(
        _tc_mid_body,
        grid=(GRID,),
        in_specs=[_row_spec(), _row_spec(3), _full_spec((3, D)),
                  _full_spec((1, D)), _full_spec((1, D)),
                  _full_spec((D, D)), _full_spec((3, D))],
        out_specs=(_row_spec(), _row_spec()),
        out_shape=(jax.ShapeDtypeStruct((N_PAD, D), jnp.float32),
                   jax.ShapeDtypeStruct((N_PAD, D), jnp.float32)),
    )(s, posp, wp_prev, g, b, wh, wp)


def _tc_final_body(s_ref, pos_ref, wpp_ref, g_ref, b_ref,
                   e0_ref, e1_ref, e2_ref, e3_ref,
                   w0_ref, b0_ref, w1_ref, b1_ref, w2_ref, b2_ref,
                   w3_ref, b3_ref, sc_ref, out_ref):
    s = s_ref[...]
    bb = _pos_mm(pos_ref[...], wpp_ref[...])
    o = jnp.where(s != -jnp.inf, s - bb, 0.0)
    emb4 = jax.nn.relu(_ln(o, g_ref[...], b_ref[...]))
    z = jnp.concatenate(
        [e0_ref[...], e1_ref[...], e2_ref[...], e3_ref[...], emb4], axis=-1)
    mu = jnp.mean(z, axis=-1, keepdims=True)
    var = jnp.mean((z - mu) ** 2, axis=-1, keepdims=True)
    z = (z - mu) * lax.rsqrt(var + 1e-5)
    z = jax.nn.relu(jnp.dot(z, w0_ref[...], preferred_element_type=jnp.float32)
                    + b0_ref[...])
    z = jax.nn.relu(jnp.dot(z, w1_ref[...], preferred_element_type=jnp.float32)
                    + b1_ref[...])
    z = jax.nn.relu(jnp.dot(z, w2_ref[...], preferred_element_type=jnp.float32)
                    + b2_ref[...])
    z = (jnp.dot(z, w3_ref[...], preferred_element_type=jnp.float32)
         + b3_ref[...])
    out_ref[...] = z * sc_ref[...]


def _tc_final(s4, posp, wp4, g4, b4, embs, mw, mb, scale):
    return pl.pallas_call(
        _tc_final_body,
        grid=(GRID,),
        in_specs=[_row_spec(), _row_spec(3), _full_spec((3, D)),
                  _full_spec((1, D)), _full_spec((1, D)),
                  _row_spec(), _row_spec(), _row_spec(), _row_spec(),
                  _full_spec((320, 128)), _full_spec((1, 128)),
                  _full_spec((128, 128)), _full_spec((1, 128)),
                  _full_spec((128, 64)), _full_spec((1, 64)),
                  _full_spec((64, 2)), _full_spec((1, 2)),
                  _full_spec((1, 2))],
        out_specs=_row_spec(2),
        out_shape=jax.ShapeDtypeStruct((N_PAD, 2), jnp.float32),
    )(s4, posp, wp4, g4, b4, embs[0], embs[1], embs[2], embs[3],
      mw[0], mb[0], mw[1], mb[1], mw[2], mb[2], mw[3], mb[3], scale)


def kernel(x, pos, edge_index, params):
    src = edge_index[0]
    dst = edge_index[1]
    srcp = jnp.concatenate([src, jnp.zeros((E_IN - E,), jnp.int32)])
    dstp = jnp.concatenate([dst, jnp.full((E_IN - E,), 2 * N_PAD, jnp.int32)])
    xp = jnp.pad(x, ((0, N_PAD - N), (0, 0)))
    posp = jnp.pad(pos, ((0, N_PAD - N), (0, 0)))

    srcs, dls, cnts = _build_edge_lists(srcp, dstp)

    ws = [params["w%d" % i] for i in range(5)]
    whs = [ws[0][:1]] + [w[:D] for w in ws[1:]]
    wps = [w[-3:] for w in ws]
    lgs = [params["ln_g%d" % i].reshape(1, D) for i in range(5)]
    lbs = [params["ln_b%d" % i].reshape(1, D) for i in range(5)]

    a = _tc_first(xp, posp, whs[0], wps[0])
    s = _segment_max(a, srcs, dls, cnts)
    embs = []
    for i in range(1, 5):
        emb, a = _tc_mid(s, posp, wps[i - 1], lgs[i - 1], lbs[i - 1],
                         whs[i], wps[i])
        embs.append(emb)
        s = _segment_max(a, srcs, dls, cnts)

    mw = [params["mlp_w%d" % j] for j in range(4)]
    mb = [params["mlp_b%d" % j].reshape(1, -1) for j in range(4)]
    out = _tc_final(s, posp, wps[4], lgs[4], lbs[4], embs, mw, mb,
                    params["scale"].reshape(1, 2))
    return out[:N]
